# Initial kernel scaffold; baseline (speedup 1.0000x reference)
import jax
import jax.numpy as jnp
from jax.experimental import pallas as pl


def kernel(data, W1, b1, W2, b2, W3, b3, W4, b4, Wfc, bfc):
    raise NotImplementedError("write your pallas kernel here")


# speedup vs baseline: 1.8062x; 1.8062x over previous
import jax
import jax.numpy as jnp
from jax import lax
from jax.experimental import pallas as pl
from jax.experimental.pallas import tpu as pltpu
from jax.experimental.pallas import tpu_sc as plsc

_PARENT_IDX_COL = 19
_BLOCK_START_COL = 28
_IND_KEEP = (0, 1, 2, 4, 5, 10)

_D = 64
_DP = 128

_NC = 2
_NS = 16
_NW = _NC * _NS

_TC_CH = 2000
_SC_CH = 640
_SC_NCH = 10
_ROWS_PER_W = _SC_CH * _SC_NCH
_M_PAD = _ROWS_PER_W * _NW


def _feat1_body(data_ref, w_ref, b_ref, out_ref):
    out_ref[...] = (
        jnp.dot(data_ref[...], w_ref[...], preferred_element_type=jnp.float32)
        + b_ref[...]
    )


def _layer_body(feat_ref, pf_ref, bs_ref, wa_ref, wb_ref, b_ref, out_ref):
    feat = feat_ref[...]
    pf = jnp.where(bs_ref[...] == 1.0, 0.0, pf_ref[...])
    acc = jnp.dot(feat, wa_ref[...], preferred_element_type=jnp.float32)
    acc = acc + jnp.dot(pf, wb_ref[...], preferred_element_type=jnp.float32)
    out_ref[...] = jnp.maximum(acc + b_ref[...] + feat, 0.0)


def _final_body(feat_ref, pf_ref, bs_ref, wa_ref, wb_ref, b_ref, wfc_ref,
                bfc_ref, out_ref):
    feat = feat_ref[...]
    pf = jnp.where(bs_ref[...] == 1.0, 0.0, pf_ref[...])
    acc = jnp.dot(feat, wa_ref[...], preferred_element_type=jnp.float32)
    acc = acc + jnp.dot(pf, wb_ref[...], preferred_element_type=jnp.float32)
    f4 = jnp.maximum(acc + b_ref[...] + feat, 0.0)
    out_ref[...] = (
        jnp.dot(f4, wfc_ref[...], preferred_element_type=jnp.float32)
        + bfc_ref[...]
    )


def _row_spec(ch, d):
    return pl.BlockSpec((ch, d), lambda i: (i, 0))


def _full_spec(shape):
    return pl.BlockSpec(shape, lambda i: (0,) * len(shape))


def _gather_sc_body(table_hbm, idx_hbm, out_hbm, idx_v, rows_v, sem):
    wid = lax.axis_index("s") * _NC + lax.axis_index("c")
    for c in range(_SC_NCH):
        base = wid * _ROWS_PER_W + c * _SC_CH
        pltpu.sync_copy(idx_hbm.at[pl.ds(base, _SC_CH)], idx_v)
        pltpu.async_copy(table_hbm.at[idx_v], rows_v, sem).wait()
        pltpu.sync_copy(rows_v, out_hbm.at[pl.ds(base, _SC_CH)])


def _make_sc_gather():
    mesh = plsc.VectorSubcoreMesh(
        core_axis_name="c", subcore_axis_name="s",
        num_cores=_NC, num_subcores=_NS)
    return pl.kernel(
        _gather_sc_body,
        out_type=jax.ShapeDtypeStruct((_M_PAD, _DP), jnp.float32),
        mesh=mesh,
        scratch_types=[
            pltpu.VMEM((_SC_CH,), jnp.int32),
            pltpu.VMEM((_SC_CH, _DP), jnp.float32),
            pltpu.SemaphoreType.DMA,
        ],
    )


def _pad_lanes(w):
    return jnp.concatenate(
        [w, jnp.zeros((_DP - w.shape[0], w.shape[1]), w.dtype)], axis=0)


def kernel(data, W1, b1, W2, b2, W3, b3, W4, b4, Wfc, bfc):
    Bs, Ns, Fs = data.shape
    M = Bs * Ns
    n_ch = M // _TC_CH

    flat = data.reshape(M, Fs)

    pidx = flat[:, _PARENT_IDX_COL].astype(jnp.int32)
    pidx = pidx + jnp.repeat(jnp.arange(Bs, dtype=jnp.int32) * Ns, Ns)
    pidx_pad = jnp.pad(pidx, (0, _M_PAD - M))

    bs_col = flat[:, _BLOCK_START_COL:_BLOCK_START_COL + 1]

    w1_full = jnp.zeros((Fs, _DP), jnp.float32)
    w1_full = w1_full.at[jnp.array(_IND_KEEP), :_D].set(W1)
    b1_pad = jnp.zeros((1, _DP), jnp.float32).at[:, :_D].set(b1)

    feat1 = pl.pallas_call(
        _feat1_body,
        grid=(n_ch,),
        in_specs=[
            _row_spec(_TC_CH, Fs),
            _full_spec((Fs, _DP)),
            _full_spec((1, _DP)),
        ],
        out_specs=_row_spec(_TC_CH, _DP),
        out_shape=jax.ShapeDtypeStruct((M, _DP), jnp.float32),
    )(flat, w1_full, b1_pad)

    gather = _make_sc_gather()

    def layer(feat, W, b, body, extra, out_d):
        pf = gather(feat, pidx_pad)
        return pl.pallas_call(
            body,
            grid=(n_ch,),
            in_specs=[
                _row_spec(_TC_CH, _DP),
                _row_spec(_TC_CH, _DP),
                _row_spec(_TC_CH, 1),
                _full_spec((_DP, out_d if body is _layer_body else _DP)),
                _full_spec((_DP, out_d if body is _layer_body else _DP)),
                _full_spec((1, out_d if body is _layer_body else _DP)),
            ] + [_full_spec(e.shape) for e in extra],
            out_specs=_row_spec(_TC_CH, out_d),
            out_shape=jax.ShapeDtypeStruct((M, out_d), jnp.float32),
        )(feat, pf, bs_col, *extra_weights(W, b), *extra)

    def extra_weights(W, b):
        wa = jnp.zeros((_DP, _DP), jnp.float32).at[:_D, :_D].set(W[:_D])
        wb = jnp.zeros((_DP, _DP), jnp.float32).at[:_D, :_D].set(W[_D:])
        bp = jnp.zeros((1, _DP), jnp.float32).at[:, :_D].set(b)
        return wa, wb, bp

    feat2 = layer(feat1, W2, b2, _layer_body, [], _DP)
    feat3 = layer(feat2, W3, b3, _layer_body, [], _DP)
    out = layer(feat3, W4, b4, _final_body,
                [_pad_lanes(Wfc), bfc.reshape(1, Wfc.shape[1])],
                Wfc.shape[1])

    return out.reshape(Bs, Ns, Wfc.shape[1])


# speedup vs baseline: 1.8788x; 1.0402x over previous
import jax
import jax.numpy as jnp
from jax import lax
from jax.experimental import pallas as pl
from jax.experimental.pallas import tpu as pltpu
from jax.experimental.pallas import tpu_sc as plsc

_PARENT_IDX_COL = 19
_BLOCK_START_COL = 28
_IND_KEEP = (0, 1, 2, 4, 5, 10)

_D = 64
_DP = 128

_NC = 2
_NS = 16
_NW = _NC * _NS

_TC_CH = 2000
_SC_CH = 400
_SC_NCH = 16
_ROWS_PER_W = _SC_CH * _SC_NCH
_M_PAD = _ROWS_PER_W * _NW


def _feat1_body(data_ref, w_ref, b_ref, out_ref):
    out_ref[...] = (
        jnp.dot(data_ref[...], w_ref[...], preferred_element_type=jnp.float32)
        + b_ref[...]
    )


def _layer_body(feat_ref, pf_ref, bs_ref, wa_ref, wb_ref, b_ref, out_ref):
    feat = feat_ref[...]
    pf = jnp.where(bs_ref[...] == 1.0, 0.0, pf_ref[...])
    acc = jnp.dot(feat, wa_ref[...], preferred_element_type=jnp.float32)
    acc = acc + jnp.dot(pf, wb_ref[...], preferred_element_type=jnp.float32)
    out_ref[...] = jnp.maximum(acc + b_ref[...] + feat, 0.0)


def _final_body(feat_ref, pf_ref, bs_ref, wa_ref, wb_ref, b_ref, wfc_ref,
                bfc_ref, out_ref):
    feat = feat_ref[...]
    pf = jnp.where(bs_ref[...] == 1.0, 0.0, pf_ref[...])
    acc = jnp.dot(feat, wa_ref[...], preferred_element_type=jnp.float32)
    acc = acc + jnp.dot(pf, wb_ref[...], preferred_element_type=jnp.float32)
    f4 = jnp.maximum(acc + b_ref[...] + feat, 0.0)
    out_ref[...] = (
        jnp.dot(f4, wfc_ref[...], preferred_element_type=jnp.float32)
        + bfc_ref[...]
    )


def _row_spec(ch, d):
    return pl.BlockSpec((ch, d), lambda i: (i, 0))


def _full_spec(shape):
    return pl.BlockSpec(shape, lambda i: (0,) * len(shape))


def _gather_sc_body(table_hbm, idx_hbm, out_hbm, idx_v, rows0, rows1,
                    gsem0, gsem1, ssem0, ssem1):
    wid = lax.axis_index("s") * _NC + lax.axis_index("c")
    base = wid * _ROWS_PER_W
    pltpu.sync_copy(idx_hbm.at[pl.ds(base, _ROWS_PER_W)], idx_v)

    bufs = (rows0, rows1)
    gsems = (gsem0, gsem1)
    ssems = (ssem0, ssem1)
    gath = [None, None]
    scat = [None, None]

    def start_gather(c, b):
        gath[b] = pltpu.async_copy(
            table_hbm.at[idx_v.at[pl.ds(c * _SC_CH, _SC_CH)]],
            bufs[b], gsems[b])

    start_gather(0, 0)
    for c in range(_SC_NCH):
        b = c & 1
        nb = 1 - b
        if c + 1 < _SC_NCH:
            if scat[nb] is not None:
                scat[nb].wait()
            start_gather(c + 1, nb)
        gath[b].wait()
        scat[b] = pltpu.async_copy(
            bufs[b], out_hbm.at[pl.ds(base + c * _SC_CH, _SC_CH)], ssems[b])
    for s in scat:
        s.wait()


def _make_sc_gather():
    mesh = plsc.VectorSubcoreMesh(
        core_axis_name="c", subcore_axis_name="s",
        num_cores=_NC, num_subcores=_NS)
    return pl.kernel(
        _gather_sc_body,
        out_type=jax.ShapeDtypeStruct((_M_PAD, _DP), jnp.float32),
        mesh=mesh,
        scratch_types=[
            pltpu.VMEM((_ROWS_PER_W,), jnp.int32),
            pltpu.VMEM((_SC_CH, _DP), jnp.float32),
            pltpu.VMEM((_SC_CH, _DP), jnp.float32),
            pltpu.SemaphoreType.DMA,
            pltpu.SemaphoreType.DMA,
            pltpu.SemaphoreType.DMA,
            pltpu.SemaphoreType.DMA,
        ],
    )


def _pad_lanes(w):
    return jnp.concatenate(
        [w, jnp.zeros((_DP - w.shape[0], w.shape[1]), w.dtype)], axis=0)


def kernel(data, W1, b1, W2, b2, W3, b3, W4, b4, Wfc, bfc):
    Bs, Ns, Fs = data.shape
    M = Bs * Ns
    n_ch = M // _TC_CH

    flat = data.reshape(M, Fs)

    pidx = flat[:, _PARENT_IDX_COL].astype(jnp.int32)
    pidx = pidx + jnp.repeat(jnp.arange(Bs, dtype=jnp.int32) * Ns, Ns)
    pidx_pad = jnp.pad(pidx, (0, _M_PAD - M))

    bs_col = flat[:, _BLOCK_START_COL:_BLOCK_START_COL + 1]

    w1_full = jnp.zeros((Fs, _DP), jnp.float32)
    w1_full = w1_full.at[jnp.array(_IND_KEEP), :_D].set(W1)
    b1_pad = jnp.zeros((1, _DP), jnp.float32).at[:, :_D].set(b1)

    feat1 = pl.pallas_call(
        _feat1_body,
        grid=(n_ch,),
        in_specs=[
            _row_spec(_TC_CH, Fs),
            _full_spec((Fs, _DP)),
            _full_spec((1, _DP)),
        ],
        out_specs=_row_spec(_TC_CH, _DP),
        out_shape=jax.ShapeDtypeStruct((M, _DP), jnp.float32),
    )(flat, w1_full, b1_pad)

    gather = _make_sc_gather()

    def layer(feat, W, b, body, extra, out_d):
        pf = gather(feat, pidx_pad)
        return pl.pallas_call(
            body,
            grid=(n_ch,),
            in_specs=[
                _row_spec(_TC_CH, _DP),
                _row_spec(_TC_CH, _DP),
                _row_spec(_TC_CH, 1),
                _full_spec((_DP, out_d if body is _layer_body else _DP)),
                _full_spec((_DP, out_d if body is _layer_body else _DP)),
                _full_spec((1, out_d if body is _layer_body else _DP)),
            ] + [_full_spec(e.shape) for e in extra],
            out_specs=_row_spec(_TC_CH, out_d),
            out_shape=jax.ShapeDtypeStruct((M, out_d), jnp.float32),
        )(feat, pf, bs_col, *extra_weights(W, b), *extra)

    def extra_weights(W, b):
        wa = jnp.zeros((_DP, _DP), jnp.float32).at[:_D, :_D].set(W[:_D])
        wb = jnp.zeros((_DP, _DP), jnp.float32).at[:_D, :_D].set(W[_D:])
        bp = jnp.zeros((1, _DP), jnp.float32).at[:, :_D].set(b)
        return wa, wb, bp

    feat2 = layer(feat1, W2, b2, _layer_body, [], _DP)
    feat3 = layer(feat2, W3, b3, _layer_body, [], _DP)
    out = layer(feat3, W4, b4, _final_body,
                [_pad_lanes(Wfc), bfc.reshape(1, Wfc.shape[1])],
                Wfc.shape[1])

    return out.reshape(Bs, Ns, Wfc.shape[1])
